# baseline (speedup 1.0000x reference)
import functools

import jax
import jax.numpy as jnp
from jax import lax
from jax.experimental import pallas as pl
from jax.experimental.pallas import tpu as pltpu
from jax.experimental.pallas import tpu_sc as plsc

VOCAB = 100000
N_INDEX = 26
EMBED_DIM = 32
BATCH = 16384

CHUNK = 4096
NCHUNK = BATCH // CHUNK
UNROLL = 16


@functools.lru_cache(maxsize=1)
def _build():
    info = plsc.get_sparse_core_info()
    nc, ns, nl = info.num_cores, info.num_subcores, info.num_lanes
    mesh = plsc.VectorSubcoreMesh(core_axis_name="c", subcore_axis_name="s")

    @functools.partial(
        pl.kernel,
        mesh=mesh,
        out_type=jax.ShapeDtypeStruct((N_INDEX, EMBED_DIM, BATCH),
                                      jnp.float32),
        compiler_params=pltpu.CompilerParams(needs_layout_passes=False),
        scratch_types=[
            pltpu.VMEM((VOCAB,), jnp.float32),
            pltpu.VMEM((CHUNK,), jnp.int32),
            pltpu.VMEM((CHUNK,), jnp.int32),
            pltpu.VMEM((CHUNK,), jnp.float32),
            pltpu.VMEM((CHUNK,), jnp.float32),
            pltpu.VMEM_SHARED((2, BATCH), jnp.int32),
            pltpu.SemaphoreType.DMA,
            pltpu.SemaphoreType.DMA,
            pltpu.SemaphoreType.DMA,
            pltpu.SemaphoreType.DMA,
        ],
    )
    def embed_kernel(ids_hbm, tab_hbm, out_hbm,
                     row_v, ids_a, ids_b, out_a, out_b, ids_sh,
                     row_sem, ids_sem, stage_sem, out_sem):
        d = lax.axis_index("s") * nc + lax.axis_index("c")
        sid = lax.axis_index("s")
        ids_bufs = (ids_a, ids_b)
        out_bufs = (out_a, out_b)

        def fire_row(i):
            pltpu.async_copy(tab_hbm.at[i, d, :], row_v, row_sem)

        def wait_row():
            pltpu.make_async_copy(tab_hbm.at[0, 0, :], row_v, row_sem).wait()

        def fire_stage(i):
            @pl.when(sid == 0)
            def _():
                pltpu.async_copy(ids_hbm.at[i], ids_sh.at[lax.rem(i, 2)],
                                 stage_sem)

        def wait_stage_and_barrier():
            @pl.when(sid == 0)
            def _():
                pltpu.make_async_copy(ids_hbm.at[0], ids_sh.at[0],
                                      stage_sem).wait()
            plsc.subcore_barrier()

        fire_row(0)
        fire_stage(0)
        wait_stage_and_barrier()
        fire_stage(1)
        pltpu.async_copy(ids_sh.at[0, pl.ds(0, CHUNK)], ids_a, ids_sem)

        def table_body(i, carry):
            p = lax.rem(i, 2)
            wait_row()
            for cc in range(NCHUNK):
                ids_v = ids_bufs[cc % 2]
                out_v = out_bufs[cc % 2]
                pltpu.make_async_copy(
                    ids_sh.at[0, pl.ds(0, CHUNK)], ids_v, ids_sem).wait()
                if cc < NCHUNK - 1:
                    pltpu.async_copy(
                        ids_sh.at[p, pl.ds((cc + 1) * CHUNK, CHUNK)],
                        ids_bufs[(cc + 1) % 2], ids_sem)
                else:
                    @pl.when(i < N_INDEX - 1)
                    def _():
                        wait_stage_and_barrier()
                        pltpu.async_copy(
                            ids_sh.at[1 - p, pl.ds(0, CHUNK)],
                            ids_bufs[0], ids_sem)

                    @pl.when(i < N_INDEX - 2)
                    def _():
                        fire_stage(i + 2)
                @pl.when(jnp.logical_or(i > 0, cc >= 2))
                def _():
                    pltpu.make_async_copy(
                        out_v, out_hbm.at[0, d, pl.ds(0, CHUNK)],
                        out_sem).wait()

                def gather_body(k, carry2):
                    base = k * (nl * UNROLL)
                    for u in range(UNROLL):
                        off = base + u * nl
                        idx = ids_v[pl.ds(off, nl)]
                        out_v[pl.ds(off, nl)] = plsc.load_gather(row_v, [idx])
                    return carry2

                lax.fori_loop(0, CHUNK // (nl * UNROLL), gather_body, 0)
                pltpu.async_copy(
                    out_v, out_hbm.at[i, d, pl.ds(cc * CHUNK, CHUNK)],
                    out_sem)
            @pl.when(i < N_INDEX - 1)
            def _():
                fire_row(i + 1)
            return carry

        lax.fori_loop(0, N_INDEX, table_body, 0)
        for b in range(2):
            pltpu.make_async_copy(
                out_bufs[b], out_hbm.at[0, d, pl.ds(0, CHUNK)],
                out_sem).wait()

    return embed_kernel


def kernel(input_ids, tables):
    embed_kernel = _build()
    ids_t = input_ids.T
    tab_t = jnp.transpose(tables, (0, 2, 1))
    out_t = embed_kernel(ids_t, tab_t)
    return jnp.transpose(out_t, (2, 0, 1))


# speedup vs baseline: 1.1171x; 1.1171x over previous
import functools

import jax
import jax.numpy as jnp
from jax import lax
from jax.experimental import pallas as pl
from jax.experimental.pallas import tpu as pltpu
from jax.experimental.pallas import tpu_sc as plsc

VOCAB = 100000
N_INDEX = 26
EMBED_DIM = 32
BATCH = 16384

CHUNK = 4096
NCHUNK = BATCH // CHUNK
UNROLL = 16


@functools.lru_cache(maxsize=1)
def _build():
    info = plsc.get_sparse_core_info()
    nc, ns, nl = info.num_cores, info.num_subcores, info.num_lanes
    mesh = plsc.VectorSubcoreMesh(core_axis_name="c", subcore_axis_name="s")

    @functools.partial(
        pl.kernel,
        mesh=mesh,
        out_type=jax.ShapeDtypeStruct((N_INDEX, EMBED_DIM, BATCH),
                                      jnp.float32),
        compiler_params=pltpu.CompilerParams(needs_layout_passes=False),
        scratch_types=[
            pltpu.VMEM((VOCAB,), jnp.float32),
            pltpu.VMEM((CHUNK,), jnp.int32),
            pltpu.VMEM((CHUNK,), jnp.int32),
            pltpu.VMEM((CHUNK,), jnp.float32),
            pltpu.VMEM((CHUNK,), jnp.float32),
            pltpu.SemaphoreType.DMA,
            pltpu.SemaphoreType.DMA,
            pltpu.SemaphoreType.DMA,
        ],
    )
    def embed_kernel(ids_hbm, tab_hbm, out_hbm,
                     row_v, ids_a, ids_b, out_a, out_b,
                     row_sem, ids_sem, out_sem):
        d = lax.axis_index("s") * nc + lax.axis_index("c")
        ids_bufs = (ids_a, ids_b)
        out_bufs = (out_a, out_b)

        def fire_row(i):
            pltpu.async_copy(tab_hbm.at[i, d, :], row_v, row_sem)

        def wait_row():
            pltpu.make_async_copy(tab_hbm.at[0, 0, :], row_v, row_sem).wait()

        fire_row(0)
        pltpu.async_copy(ids_hbm.at[0, pl.ds(0, CHUNK)], ids_a, ids_sem)

        def table_body(i, carry):
            wait_row()
            for cc in range(NCHUNK):
                ids_v = ids_bufs[cc % 2]
                out_v = out_bufs[cc % 2]
                pltpu.make_async_copy(
                    ids_hbm.at[0, pl.ds(0, CHUNK)], ids_v, ids_sem).wait()
                if cc < NCHUNK - 1:
                    pltpu.async_copy(
                        ids_hbm.at[i, pl.ds((cc + 1) * CHUNK, CHUNK)],
                        ids_bufs[(cc + 1) % 2], ids_sem)
                else:
                    @pl.when(i < N_INDEX - 1)
                    def _():
                        pltpu.async_copy(
                            ids_hbm.at[i + 1, pl.ds(0, CHUNK)],
                            ids_bufs[0], ids_sem)
                @pl.when(jnp.logical_or(i > 0, cc >= 2))
                def _():
                    pltpu.make_async_copy(
                        out_v, out_hbm.at[0, d, pl.ds(0, CHUNK)],
                        out_sem).wait()

                def gather_body(k, carry2):
                    base = k * (nl * UNROLL)
                    idxs = [ids_v[pl.ds(base + u * nl, nl)]
                            for u in range(UNROLL)]
                    vals = [plsc.load_gather(row_v, [idxs[u]])
                            for u in range(UNROLL)]
                    for u in range(UNROLL):
                        out_v[pl.ds(base + u * nl, nl)] = vals[u]
                    return carry2

                lax.fori_loop(0, CHUNK // (nl * UNROLL), gather_body, 0)
                pltpu.async_copy(
                    out_v, out_hbm.at[i, d, pl.ds(cc * CHUNK, CHUNK)],
                    out_sem)
            @pl.when(i < N_INDEX - 1)
            def _():
                fire_row(i + 1)
            return carry

        lax.fori_loop(0, N_INDEX, table_body, 0)
        for b in range(2):
            pltpu.make_async_copy(
                out_bufs[b], out_hbm.at[0, d, pl.ds(0, CHUNK)],
                out_sem).wait()

    return embed_kernel


def kernel(input_ids, tables):
    embed_kernel = _build()
    ids_t = input_ids.T
    tab_t = jnp.transpose(tables, (0, 2, 1))
    out_t = embed_kernel(ids_t, tab_t)
    return jnp.transpose(out_t, (2, 0, 1))
